# Initial kernel scaffold; baseline (speedup 1.0000x reference)
import jax
import jax.numpy as jnp
from jax.experimental import pallas as pl


def kernel(preds, gt, epoch, device):
    raise NotImplementedError("write your pallas kernel here")


# speedup vs baseline: 14.3899x; 14.3899x over previous
import functools

import jax
import jax.numpy as jnp
from jax import lax
from jax.experimental import pallas as pl
from jax.experimental.pallas import tpu as pltpu

_START_WARM = 12
_TOP_P = 0.15


def _ce_loss_kernel(preds_ref, gt_ref, loss_ref, *, num_classes):
    g = gt_ref[0]
    m = preds_ref[0, 0]
    for c in range(1, num_classes):
        m = jnp.maximum(m, preds_ref[0, c])
    s = jnp.zeros_like(m)
    picked = jnp.zeros_like(m)
    for c in range(num_classes):
        xc = preds_ref[0, c]
        s = s + jnp.exp(xc - m)
        picked = picked + jnp.where(g == c, xc, 0.0)
    loss_ref[0] = (m - picked) + jnp.log(s)


def _select_kernel(loss_ref, out_ref, *, k, n):
    def body(_, carry):
        lo, hi = carry
        mid = lo + ((hi - lo + 1) >> 1)
        bits = lax.bitcast_convert_type(loss_ref[...], jnp.int32)
        cnt = jnp.sum((bits >= mid).astype(jnp.int32))
        ge = cnt >= k
        lo2 = jnp.where(ge, mid, lo)
        hi2 = jnp.where(ge, hi, mid - 1)
        return lo2, hi2

    lo, _ = lax.fori_loop(
        0, 31, body, (jnp.int32(0), jnp.int32(0x7F800000)))

    x = loss_ref[...]
    bits = lax.bitcast_convert_type(x, jnp.int32)
    strict = bits > lo
    sum_gt = jnp.sum(jnp.where(strict, x, 0.0))
    cnt_gt = jnp.sum(strict.astype(jnp.int32))
    sum_all = jnp.sum(x)
    v = lax.bitcast_convert_type(lo, jnp.float32)
    topk_sum = sum_gt + (k - cnt_gt).astype(jnp.float32) * v
    out_ref[0, 0] = topk_sum / jnp.float32(k)
    out_ref[0, 1] = sum_all / jnp.float32(n)


def kernel(preds, gt, epoch, device):
    b, c, h, w = preds.shape
    n = b * h * w
    k = int(n * _TOP_P)
    br = 64

    loss = pl.pallas_call(
        functools.partial(_ce_loss_kernel, num_classes=c),
        grid=(b, h // br),
        in_specs=[
            pl.BlockSpec((1, c, br, w), lambda i, r: (i, 0, r, 0)),
            pl.BlockSpec((1, br, w), lambda i, r: (i, r, 0)),
        ],
        out_specs=pl.BlockSpec((1, br, w), lambda i, r: (i, r, 0)),
        out_shape=jax.ShapeDtypeStruct((b, h, w), jnp.float32),
    )(preds, gt)

    loss2 = loss.reshape(n // 1024, 1024)
    means = pl.pallas_call(
        functools.partial(_select_kernel, k=k, n=n),
        out_specs=pl.BlockSpec(memory_space=pltpu.SMEM),
        out_shape=jax.ShapeDtypeStruct((1, 2), jnp.float32),
    )(loss2)

    out = jnp.where(epoch < _START_WARM, means[0, 1], means[0, 0])
    return out + jnp.asarray(device * 0).astype(out.dtype)
